# Initial kernel scaffold; baseline (speedup 1.0000x reference)
import jax
import jax.numpy as jnp
from jax.experimental import pallas as pl


def kernel(x, edge_index):
    raise NotImplementedError("write your pallas kernel here")


# speedup vs baseline: 6.7782x; 6.7782x over previous
import functools

import jax
import jax.numpy as jnp
from jax import lax
from jax.experimental import pallas as pl
from jax.experimental.pallas import tpu as pltpu
from jax.experimental.pallas import tpu_sc as plsc

N_NODES = 10000
N_EDGES = 320000
D_FEAT = 128

_INFO = plsc.get_sparse_core_info()
NC = _INFO.num_cores
NS = _INFO.num_subcores
NW = NC * NS

CHUNK = 128
N_CHUNKS = N_EDGES // CHUNK
BASE_CHUNKS = N_CHUNKS // NW
EXTRA = N_CHUNKS - BASE_CHUNKS * NW

ROW_BLOCKS = N_NODES // 8
RB_BASE = ROW_BLOCKS // NS
RB_EXTRA = ROW_BLOCKS - RB_BASE * NS
ROWS_BASE = RB_BASE * 8


@functools.partial(
    pl.kernel,
    mesh=plsc.VectorSubcoreMesh(core_axis_name="c", subcore_axis_name="s"),
    out_type=jax.ShapeDtypeStruct((NC, N_NODES, D_FEAT), jnp.float32),
    scratch_types=[
        pltpu.VMEM((CHUNK,), jnp.int32),
        pltpu.VMEM((CHUNK,), jnp.int32),
        pltpu.VMEM((CHUNK, D_FEAT), jnp.float32),
        pltpu.VMEM_SHARED((N_NODES, D_FEAT), jnp.float32),
        pltpu.SemaphoreType.DMA,
    ],
)
def _sc_scatter_sum(x_hbm, src_hbm, dst_hbm, out_hbm,
                    src_v, dst_v, rows_v, acc, sem):
    c = lax.axis_index("c")
    s = lax.axis_index("s")
    wid = s * NC + c

    zero16 = jnp.zeros((16,), jnp.float32)

    def _zero_row(r, carry):
        for j in range(D_FEAT // 16):
            rows_v[r, pl.ds(j * 16, 16)] = zero16
        return carry

    lax.fori_loop(0, CHUNK, _zero_row, 0)

    row0 = (s * RB_BASE + jnp.minimum(s, RB_EXTRA)) * 8
    has_extra = s < RB_EXTRA
    n_full = ROWS_BASE // CHUNK
    tail = ROWS_BASE - n_full * CHUNK
    for k in range(n_full):
        pltpu.sync_copy(rows_v, acc.at[pl.ds(row0 + k * CHUNK, CHUNK)])
    pltpu.sync_copy(rows_v.at[pl.ds(0, tail)],
                    acc.at[pl.ds(row0 + n_full * CHUNK, tail)])

    @pl.when(has_extra)
    def _zero_extra():
        pltpu.sync_copy(rows_v.at[pl.ds(0, 8)],
                        acc.at[pl.ds(row0 + ROWS_BASE, 8)])

    plsc.subcore_barrier()

    start = wid * BASE_CHUNKS + jnp.minimum(wid, EXTRA)
    n_chunks = BASE_CHUNKS + jnp.where(wid < EXTRA, 1, 0)

    def _chunk_body(i, carry):
        base = (start + i) * CHUNK
        pltpu.sync_copy(src_hbm.at[pl.ds(base, CHUNK)], src_v)
        pltpu.sync_copy(dst_hbm.at[pl.ds(base, CHUNK)], dst_v)
        pltpu.async_copy(x_hbm.at[src_v], rows_v, sem).wait()
        pltpu.sync_copy(rows_v, acc.at[dst_v], add=True)
        return carry

    lax.fori_loop(0, n_chunks, _chunk_body, 0)
    plsc.subcore_barrier()

    pltpu.sync_copy(acc.at[pl.ds(row0, ROWS_BASE)],
                    out_hbm.at[c, pl.ds(row0, ROWS_BASE)])

    @pl.when(has_extra)
    def _write_extra():
        pltpu.sync_copy(acc.at[pl.ds(row0 + ROWS_BASE, 8)],
                        out_hbm.at[c, pl.ds(row0 + ROWS_BASE, 8)])


def _combine_body(p_ref, o_ref):
    o_ref[...] = p_ref[0] + p_ref[1]


_ROW_BLOCK = 1000

_combine = pl.pallas_call(
    _combine_body,
    out_shape=jax.ShapeDtypeStruct((N_NODES, D_FEAT), jnp.float32),
    grid=(N_NODES // _ROW_BLOCK,),
    in_specs=[pl.BlockSpec((NC, _ROW_BLOCK, D_FEAT), lambda i: (0, i, 0))],
    out_specs=pl.BlockSpec((_ROW_BLOCK, D_FEAT), lambda i: (i, 0)),
)


def kernel(x, edge_index):
    ei = edge_index.astype(jnp.int32)
    dst = ei[0]
    src = ei[1]
    partials = _sc_scatter_sum(x, src, dst)
    return _combine(partials)


# speedup vs baseline: 10.3676x; 1.5295x over previous
import functools

import jax
import jax.numpy as jnp
from jax import lax
from jax.experimental import pallas as pl
from jax.experimental.pallas import tpu as pltpu
from jax.experimental.pallas import tpu_sc as plsc

N_NODES = 10000
N_EDGES = 320000
D_FEAT = 128

_INFO = plsc.get_sparse_core_info()
NC = _INFO.num_cores
NS = _INFO.num_subcores
NW = NC * NS

CHUNK = 128
N_CHUNKS = N_EDGES // CHUNK
BASE_CHUNKS = N_CHUNKS // NW
EXTRA = N_CHUNKS - BASE_CHUNKS * NW

ROW_BLOCKS = N_NODES // 8
RB_BASE = ROW_BLOCKS // NS
RB_EXTRA = ROW_BLOCKS - RB_BASE * NS
ROWS_BASE = RB_BASE * 8


@functools.partial(
    pl.kernel,
    mesh=plsc.VectorSubcoreMesh(core_axis_name="c", subcore_axis_name="s"),
    out_type=jax.ShapeDtypeStruct((NC, N_NODES, D_FEAT), jnp.float32),
    scratch_types=[
        pltpu.VMEM((2, CHUNK), jnp.int32),
        pltpu.VMEM((2, CHUNK), jnp.int32),
        pltpu.VMEM((2, CHUNK, D_FEAT), jnp.float32),
        pltpu.VMEM_SHARED((N_NODES, D_FEAT), jnp.float32),
        pltpu.SemaphoreType.DMA,
        pltpu.SemaphoreType.DMA,
    ],
)
def _sc_scatter_sum(x_hbm, src_hbm, dst_hbm, out_hbm,
                    src_v, dst_v, rows_v, acc, sem0, sem1):
    c = lax.axis_index("c")
    s = lax.axis_index("s")
    wid = s * NC + c
    sems = (sem0, sem1)

    zero16 = jnp.zeros((16,), jnp.float32)

    def _zero_row(r, carry):
        for j in range(D_FEAT // 16):
            rows_v[0, r, pl.ds(j * 16, 16)] = zero16
        return carry

    lax.fori_loop(0, CHUNK, _zero_row, 0)

    row0 = (s * RB_BASE + jnp.minimum(s, RB_EXTRA)) * 8
    has_extra = s < RB_EXTRA
    n_full = ROWS_BASE // CHUNK
    tail = ROWS_BASE - n_full * CHUNK
    for k in range(n_full):
        pltpu.sync_copy(rows_v.at[0], acc.at[pl.ds(row0 + k * CHUNK, CHUNK)])
    pltpu.sync_copy(rows_v.at[0, pl.ds(0, tail)],
                    acc.at[pl.ds(row0 + n_full * CHUNK, tail)])

    @pl.when(has_extra)
    def _zero_extra():
        pltpu.sync_copy(rows_v.at[0, pl.ds(0, 8)],
                        acc.at[pl.ds(row0 + ROWS_BASE, 8)])

    plsc.subcore_barrier()

    start = wid * BASE_CHUNKS + jnp.minimum(wid, EXTRA)
    n_chunks = BASE_CHUNKS + jnp.where(wid < EXTRA, 1, 0)

    def _load_idx(j, b):
        base = (start + j) * CHUNK
        pltpu.sync_copy(src_hbm.at[pl.ds(base, CHUNK)], src_v.at[b])
        pltpu.sync_copy(dst_hbm.at[pl.ds(base, CHUNK)], dst_v.at[b])

    def _fire_gather(b):
        pltpu.async_copy(x_hbm.at[src_v.at[b]], rows_v.at[b], sems[b])

    def _wait_gather(b):
        pltpu.make_async_copy(x_hbm.at[pl.ds(0, CHUNK)], rows_v.at[b],
                              sems[b]).wait()

    for b in range(2):
        _load_idx(b, b)
        _fire_gather(b)

    def _chunk_pair(j0, carry):
        for b in range(2):
            j = j0 * 2 + b
            _wait_gather(b)
            pltpu.sync_copy(rows_v.at[b], acc.at[dst_v.at[b]], add=True)

            @pl.when(j + 2 < n_chunks)
            def _refill():
                _load_idx(j + 2, b)
                _fire_gather(b)
        return carry

    lax.fori_loop(0, BASE_CHUNKS // 2, _chunk_pair, 0)

    @pl.when(n_chunks > BASE_CHUNKS)
    def _tail_chunk():
        _wait_gather(0)
        pltpu.sync_copy(rows_v.at[0], acc.at[dst_v.at[0]], add=True)

    plsc.subcore_barrier()

    pltpu.sync_copy(acc.at[pl.ds(row0, ROWS_BASE)],
                    out_hbm.at[c, pl.ds(row0, ROWS_BASE)])

    @pl.when(has_extra)
    def _write_extra():
        pltpu.sync_copy(acc.at[pl.ds(row0 + ROWS_BASE, 8)],
                        out_hbm.at[c, pl.ds(row0 + ROWS_BASE, 8)])


def _combine_body(p_ref, o_ref):
    o_ref[...] = p_ref[0] + p_ref[1]


_ROW_BLOCK = 1000

_combine = pl.pallas_call(
    _combine_body,
    out_shape=jax.ShapeDtypeStruct((N_NODES, D_FEAT), jnp.float32),
    grid=(N_NODES // _ROW_BLOCK,),
    in_specs=[pl.BlockSpec((NC, _ROW_BLOCK, D_FEAT), lambda i: (0, i, 0))],
    out_specs=pl.BlockSpec((_ROW_BLOCK, D_FEAT), lambda i: (i, 0)),
)


def kernel(x, edge_index):
    ei = edge_index.astype(jnp.int32)
    dst = ei[0]
    src = ei[1]
    partials = _sc_scatter_sum(x, src, dst)
    return _combine(partials)


# speedup vs baseline: 12.5742x; 1.2128x over previous
import functools

import jax
import jax.numpy as jnp
from jax import lax
from jax.experimental import pallas as pl
from jax.experimental.pallas import tpu as pltpu
from jax.experimental.pallas import tpu_sc as plsc

N_NODES = 10000
N_EDGES = 320000
D_FEAT = 128

_INFO = plsc.get_sparse_core_info()
NC = _INFO.num_cores
NS = _INFO.num_subcores
NW = NC * NS

CHUNK = 125
N_CHUNKS = N_EDGES // CHUNK
CHUNKS_PER_TILE = N_CHUNKS // NW
HALF = CHUNKS_PER_TILE // 2

ROW_BLOCKS = N_NODES // 8
RB_BASE = ROW_BLOCKS // NS
RB_EXTRA = ROW_BLOCKS - RB_BASE * NS
ROWS_BASE = RB_BASE * 8
ZCHUNK = 120


@functools.partial(
    pl.kernel,
    mesh=plsc.VectorSubcoreMesh(core_axis_name="c", subcore_axis_name="s"),
    out_type=jax.ShapeDtypeStruct((NC, N_NODES, D_FEAT), jnp.float32),
    scratch_types=[
        pltpu.VMEM((HALF, CHUNK), jnp.int32),
        pltpu.VMEM((HALF, CHUNK), jnp.int32),
        pltpu.VMEM((2, CHUNK, D_FEAT), jnp.float32),
        pltpu.VMEM_SHARED((N_NODES, D_FEAT), jnp.float32),
        pltpu.SemaphoreType.DMA,
        pltpu.SemaphoreType.DMA,
    ],
)
def _sc_scatter_sum(x_hbm, src_hbm, dst_hbm, dummy_hbm, out_hbm,
                    src_v, dst_v, rows_v, acc, sem0, sem1):
    c = lax.axis_index("c")
    s = lax.axis_index("s")
    wid = s * NC + c
    sems = (sem0, sem1)

    zero16 = jnp.zeros((16,), jnp.float32)

    def _zero_row(r, carry):
        for j in range(D_FEAT // 16):
            rows_v[0, r, pl.ds(j * 16, 16)] = zero16
        return carry

    lax.fori_loop(0, ZCHUNK, _zero_row, 0)

    row0 = (s * RB_BASE + jnp.minimum(s, RB_EXTRA)) * 8
    has_extra = s < RB_EXTRA
    n_full = ROWS_BASE // ZCHUNK
    tail = ROWS_BASE - n_full * ZCHUNK
    for k in range(n_full):
        pltpu.sync_copy(rows_v.at[0, pl.ds(0, ZCHUNK)],
                        acc.at[pl.ds(row0 + k * ZCHUNK, ZCHUNK)])
    pltpu.sync_copy(rows_v.at[0, pl.ds(0, tail)],
                    acc.at[pl.ds(row0 + n_full * ZCHUNK, tail)])

    @pl.when(has_extra)
    def _zero_extra():
        pltpu.sync_copy(rows_v.at[0, pl.ds(0, 8)],
                        acc.at[pl.ds(row0 + ROWS_BASE, 8)])

    plsc.subcore_barrier()

    def _fire_gather(j, b):
        pltpu.async_copy(x_hbm.at[src_v.at[j]], rows_v.at[b], sems[b])

    def _wait_gather(b):
        pltpu.make_async_copy(dummy_hbm, rows_v.at[b], sems[b]).wait()

    def _chunk_pair(j0, carry):
        for b in range(2):
            j = j0 * 2 + b
            _wait_gather(b)
            pltpu.sync_copy(rows_v.at[b], acc.at[dst_v.at[j]], add=True)

            @pl.when(j + 2 < HALF)
            def _refill():
                _fire_gather(j + 2, b)
        return carry

    for h in range(2):
        base = wid * CHUNKS_PER_TILE + h * HALF
        pltpu.sync_copy(src_hbm.at[pl.ds(base, HALF)], src_v)
        pltpu.sync_copy(dst_hbm.at[pl.ds(base, HALF)], dst_v)
        for b in range(2):
            _fire_gather(b, b)
        lax.fori_loop(0, HALF // 2, _chunk_pair, 0)

    plsc.subcore_barrier()

    pltpu.sync_copy(acc.at[pl.ds(row0, ROWS_BASE)],
                    out_hbm.at[c, pl.ds(row0, ROWS_BASE)])

    @pl.when(has_extra)
    def _write_extra():
        pltpu.sync_copy(acc.at[pl.ds(row0 + ROWS_BASE, 8)],
                        out_hbm.at[c, pl.ds(row0 + ROWS_BASE, 8)])


def _combine_body(p_ref, o_ref):
    o_ref[...] = p_ref[0] + p_ref[1]


_ROW_BLOCK = 1000

_combine = pl.pallas_call(
    _combine_body,
    out_shape=jax.ShapeDtypeStruct((N_NODES, D_FEAT), jnp.float32),
    grid=(N_NODES // _ROW_BLOCK,),
    in_specs=[pl.BlockSpec((NC, _ROW_BLOCK, D_FEAT), lambda i: (0, i, 0))],
    out_specs=pl.BlockSpec((_ROW_BLOCK, D_FEAT), lambda i: (i, 0)),
)


def kernel(x, edge_index):
    ei = edge_index.astype(jnp.int32)
    dst2d = ei[0].reshape(N_CHUNKS, CHUNK)
    src2d = ei[1].reshape(N_CHUNKS, CHUNK)
    dummy = jnp.zeros((CHUNK, D_FEAT), jnp.float32)
    partials = _sc_scatter_sum(x, src2d, dst2d, dummy)
    return _combine(partials)
